# Initial kernel scaffold; baseline (speedup 1.0000x reference)
import jax
import jax.numpy as jnp
from jax.experimental import pallas as pl


def kernel(token_ids, embedding):
    raise NotImplementedError("write your pallas kernel here")


# speedup vs baseline: 1.8397x; 1.8397x over previous
import functools

import jax
import jax.numpy as jnp
from jax import lax
from jax.experimental import pallas as pl
from jax.experimental.pallas import tpu as pltpu
from jax.experimental.pallas import tpu_sc as plsc

_B, _T = 16384, 50
_D = 64
_TOTAL = _B * _T
_NC, _NS = 2, 16
_NW = _NC * _NS
_BPW = _TOTAL // _NW
_C = 128
_NCH = _BPW // _C
_NBUF = 2

_mesh = plsc.VectorSubcoreMesh(core_axis_name="c", subcore_axis_name="s")


@functools.partial(
    pl.kernel,
    mesh=_mesh,
    out_type=jax.ShapeDtypeStruct((_TOTAL, _D), jnp.float32),
    scratch_types=[
        pltpu.VMEM((_NCH, _C), jnp.int32),
        pltpu.VMEM((_NBUF, _C, _D), jnp.float32),
        pltpu.SemaphoreType.DMA,
        pltpu.SemaphoreType.DMA,
    ],
    compiler_params=pltpu.CompilerParams(use_tc_tiling_on_sc=False),
)
def _gather_kernel(emb_hbm, idx_hbm, out_hbm, idx_v, rows_v, sem0, sem1):
    sems = (sem0, sem1)
    wid = lax.axis_index("s") * _NC + lax.axis_index("c")
    base = wid * _BPW

    pltpu.sync_copy(idx_hbm.at[wid], idx_v)

    for b in range(_NBUF):
        pltpu.async_copy(emb_hbm.at[idx_v.at[b]], rows_v.at[b], sems[b])

    def group(g, _):
        for b in range(_NBUF):
            j = g * _NBUF + b
            pltpu.make_async_copy(
                emb_hbm.at[idx_v.at[0]], rows_v.at[b], sems[b]
            ).wait()
            pltpu.sync_copy(rows_v.at[b], out_hbm.at[pl.ds(base + j * _C, _C)])
            nxt = j + _NBUF

            @pl.when(nxt < _NCH)
            def _():
                pltpu.async_copy(emb_hbm.at[idx_v.at[nxt]], rows_v.at[b], sems[b])

        return 0

    lax.fori_loop(0, _NCH // _NBUF, group, 0, unroll=False)


def kernel(token_ids, embedding):
    idx = token_ids.astype(jnp.int32).reshape(_NW, _NCH, _C)
    out = _gather_kernel(embedding, idx)
    return out.reshape(_B, _T, _D)


# speedup vs baseline: 1.8701x; 1.0165x over previous
import functools

import jax
import jax.numpy as jnp
from jax import lax
from jax.experimental import pallas as pl
from jax.experimental.pallas import tpu as pltpu
from jax.experimental.pallas import tpu_sc as plsc

_B, _T = 16384, 50
_D = 64
_NC, _NS = 2, 16
_NW = _NC * _NS
_BPW = _B // _NW
_G = 8
_NU = _BPW // _G
_NBUF = 2

_mesh = plsc.VectorSubcoreMesh(core_axis_name="c", subcore_axis_name="s")


@functools.partial(
    pl.kernel,
    mesh=_mesh,
    out_type=jax.ShapeDtypeStruct((_B, _T, _D), jnp.float32),
    scratch_types=[
        pltpu.VMEM((_BPW, _T), jnp.int32),
        pltpu.VMEM((_NBUF, _G, _T, _D), jnp.float32),
        pltpu.SemaphoreType.DMA,
        pltpu.SemaphoreType.DMA,
    ],
    compiler_params=pltpu.CompilerParams(use_tc_tiling_on_sc=False),
)
def _gather_kernel(emb_hbm, idx_hbm, out_hbm, idx_v, rows_v, sem0, sem1):
    sems = (sem0, sem1)
    wid = lax.axis_index("s") * _NC + lax.axis_index("c")
    bbase = wid * _BPW

    pltpu.sync_copy(idx_hbm.at[wid], idx_v)

    def fire(u, b):
        for q in range(_G):
            pltpu.async_copy(
                emb_hbm.at[idx_v.at[u * _G + q]], rows_v.at[b, q], sems[b]
            )

    def drain(b):
        pltpu.make_async_copy(
            out_hbm.at[pl.ds(0, _G)], rows_v.at[b], sems[b]
        ).wait()

    for b in range(_NBUF):
        fire(b, b)

    def group(g, _):
        for b in range(_NBUF):
            u = g * _NBUF + b
            drain(b)
            pltpu.sync_copy(rows_v.at[b], out_hbm.at[pl.ds(bbase + u * _G, _G)])
            nxt = u + _NBUF

            @pl.when(nxt < _NU)
            def _():
                fire(nxt, b)

        return 0

    lax.fori_loop(0, _NU // _NBUF, group, 0, unroll=False)


def kernel(token_ids, embedding):
    idx = token_ids.astype(jnp.int32).reshape(_NW, _BPW, _T)
    return _gather_kernel(embedding, idx)


# speedup vs baseline: 1.9695x; 1.0532x over previous
import functools

import jax
import jax.numpy as jnp
from jax import lax
from jax.experimental import pallas as pl
from jax.experimental.pallas import tpu as pltpu
from jax.experimental.pallas import tpu_sc as plsc

_B, _T = 16384, 50
_D = 64
_N_EMB = 1000000
_NC, _NS = 2, 16
_NW = _NC * _NS
_BPW = _B // _NW
_G = 8
_NU = _BPW // _G
_NBUF = 2

_mesh = plsc.VectorSubcoreMesh(core_axis_name="c", subcore_axis_name="s")


@functools.partial(
    pl.kernel,
    mesh=_mesh,
    out_type=jax.ShapeDtypeStruct((_B, _T, _D), jnp.float32),
    scratch_types=[
        pltpu.VMEM((_BPW, _T), jnp.int32),
        pltpu.VMEM((_NBUF, _G, _T, _D), jnp.float32),
        pltpu.SemaphoreType.DMA,
        pltpu.SemaphoreType.DMA,
    ],
    compiler_params=pltpu.CompilerParams(use_tc_tiling_on_sc=False),
)
def _gather_kernel(emb_hbm, idx_hbm, out_hbm, idx_v, rows_v, sem0, sem1):
    sems = (sem0, sem1)
    wid = lax.axis_index("s") * _NC + lax.axis_index("c")
    bbase = wid * _BPW

    pltpu.sync_copy(idx_hbm.at[wid], idx_v)

    def fire(u, b):
        for q in range(_G):
            pltpu.async_copy(
                emb_hbm.at[idx_v.at[u * _G + q]], rows_v.at[b, q], sems[b]
            )

    def drain(b):
        pltpu.make_async_copy(
            out_hbm.at[pl.ds(0, _G)], rows_v.at[b], sems[b]
        ).wait()

    for b in range(_NBUF):
        fire(b, b)

    def group(g, _):
        for b in range(_NBUF):
            u = g * _NBUF + b
            drain(b)
            pltpu.sync_copy(rows_v.at[b], out_hbm.at[pl.ds(bbase + u * _G, _G)])
            nxt = u + _NBUF

            @pl.when(nxt < _NU)
            def _():
                fire(nxt, b)

        return 0

    lax.fori_loop(0, _NU // _NBUF, group, 0, unroll=False)


def kernel(token_ids, embedding):
    emb2 = jnp.pad(embedding, ((0, 0), (0, 64))).reshape(2 * _N_EMB, _D)
    idx = (token_ids.astype(jnp.int32) * 2).reshape(_NW, _BPW, _T)
    return _gather_kernel(emb2, idx)
